# baseline (speedup 1.0000x reference)
import functools

import jax
import jax.numpy as jnp
from jax import lax
from jax.experimental import pallas as pl
from jax.experimental.pallas import tpu as pltpu
from jax.experimental.pallas import tpu_sc as plsc

N = 10000
E = 320000
D = 128

NC = 2
NS = 16
NW = NC * NS
EPW = E // NW
K = 128
E_PAD = 327680
EPP = E_PAD // NW
NCHUNK = EPP // K
ACC_ROWS = 10112
ROWS_PT = N // NS

BR = 400
GRID = N // BR

_mesh = plsc.VectorSubcoreMesh(core_axis_name="c", subcore_axis_name="s")


@functools.partial(
    pl.kernel,
    mesh=_mesh,
    compiler_params=pltpu.CompilerParams(needs_layout_passes=False),
    out_type=jax.ShapeDtypeStruct((NW, N), jnp.float32),
    scratch_types=[
        pltpu.VMEM((N,), jnp.float32),
        pltpu.VMEM((EPW,), jnp.int32),
        pltpu.VMEM((EPW,), jnp.int32),
    ],
)
def _deg_kernel(row_hbm, col_hbm, hist_hbm, hist_v, row_v, col_v):
    wid = lax.axis_index("s") * NC + lax.axis_index("c")
    base = wid * EPW
    pltpu.sync_copy(row_hbm.at[pl.ds(base, EPW)], row_v)
    pltpu.sync_copy(col_hbm.at[pl.ds(base, EPW)], col_v)

    zv = jnp.zeros((16,), jnp.float32)

    def zbody(i, t):
        hist_v[pl.ds(i * 16, 16)] = zv
        return t

    lax.fori_loop(0, N // 16, zbody, 0)

    ones = jnp.ones((16,), jnp.float32)
    zero = jnp.zeros((16,), jnp.float32)

    def ebody(i, t):
        r = row_v[pl.ds(i * 16, 16)]
        c = col_v[pl.ds(i * 16, 16)]
        val = jnp.where(r == c, zero, ones)
        plsc.addupdate_scatter(hist_v, [r], val)
        return t

    lax.fori_loop(0, EPW // 16, ebody, 0)
    pltpu.sync_copy(hist_v, hist_hbm.at[wid])


def _norm_body(x_ref, w_ref, hist_ref, m_ref):
    h = jnp.dot(x_ref[...], w_ref[...], preferred_element_type=jnp.float32)
    deg = jnp.sum(hist_ref[...], axis=1) + 1.0
    dinv = lax.rsqrt(deg)
    m_ref[...] = h * dinv[:, None]


NB = 2
PASSES = 2
CPP = NCHUNK // PASSES
GRP = CPP // NB
ZPT = ACC_ROWS // NS


@functools.partial(
    pl.kernel,
    mesh=_mesh,
    compiler_params=pltpu.CompilerParams(needs_layout_passes=False),
    out_type=jax.ShapeDtypeStruct((NC, ACC_ROWS, D), jnp.float32),
    scratch_types=[
        pltpu.VMEM_SHARED((ACC_ROWS, D), jnp.float32),
        pltpu.VMEM((CPP, K), jnp.int32),
        pltpu.VMEM((CPP, K), jnp.int32),
        [pltpu.VMEM((K, D), jnp.float32)] * NB,
        [pltpu.SemaphoreType.DMA] * NB,
        pltpu.SemaphoreType.DMA,
    ],
)
def _edge_kernel(m_hbm, row_hbm, col_hbm, outp_hbm,
                 acc_s, row_v, col_v, gbufs, gsems, isem):
    cid = lax.axis_index("c")
    sid = lax.axis_index("s")
    wid = sid * NC + cid

    idx_row = pltpu.async_copy(row_hbm.at[wid, pl.ds(0, CPP)], row_v, isem)
    idx_col = pltpu.async_copy(col_hbm.at[wid, pl.ds(0, CPP)], col_v, isem)

    zv = jnp.zeros((16,), jnp.float32)

    def zb(i, t):
        gbufs[0][i // 8, pl.ds((i % 8) * 16, 16)] = zv
        return t

    lax.fori_loop(0, K * (D // 16), zb, 0)

    def zacc(i, t):
        pltpu.sync_copy(gbufs[0], acc_s.at[pl.ds(sid * ZPT + i * K, K)])
        return t

    lax.fori_loop(0, ZPT // K, zacc, 0)
    ztail = ZPT % K
    if ztail:
        pltpu.sync_copy(gbufs[0].at[pl.ds(0, ztail)],
                        acc_s.at[pl.ds(sid * ZPT + (ZPT // K) * K, ztail)])

    idx_row.wait()
    idx_col.wait()
    plsc.subcore_barrier()

    trash = jnp.full((16,), N, jnp.int32)

    for p in range(PASSES):
        if p > 0:
            pltpu.async_copy(
                row_hbm.at[wid, pl.ds(p * CPP, CPP)], row_v, isem).wait()
            pltpu.async_copy(
                col_hbm.at[wid, pl.ds(p * CPP, CPP)], col_v, isem).wait()

        def adj(ci, t):
            for j in range(K // 16):
                r = row_v[ci, pl.ds(j * 16, 16)]
                c = col_v[ci, pl.ds(j * 16, 16)]
                col_v[ci, pl.ds(j * 16, 16)] = jnp.where(r == c, trash, c)
            return t

        lax.fori_loop(0, CPP, adj, 0)

        for b in range(NB):
            pltpu.async_copy(m_hbm.at[row_v.at[b]], gbufs[b], gsems[b])

        def group(g, t):
            ci0 = g * NB
            for b in range(NB):
                ci = ci0 + b
                pltpu.make_async_copy(m_hbm.at[row_v.at[0]], gbufs[b],
                                      gsems[b]).wait()
                pltpu.sync_copy(gbufs[b], acc_s.at[col_v.at[ci]], add=True)
                pltpu.async_copy(m_hbm.at[row_v.at[ci + NB]], gbufs[b],
                                 gsems[b])
            return t

        lax.fori_loop(0, GRP - 1, group, 0)

        for b in range(NB):
            ci = (GRP - 1) * NB + b
            pltpu.make_async_copy(m_hbm.at[row_v.at[0]], gbufs[b],
                                  gsems[b]).wait()
            pltpu.sync_copy(gbufs[b], acc_s.at[col_v.at[ci]], add=True)

    plsc.subcore_barrier()
    pltpu.sync_copy(acc_s.at[pl.ds(sid * ZPT, ZPT)],
                    outp_hbm.at[cid, pl.ds(sid * ZPT, ZPT)])


def _final_body(p_ref, m_ref, hist_ref, b_ref, o_ref):
    deg = jnp.sum(hist_ref[...], axis=1) + 1.0
    dinv = lax.rsqrt(deg)
    s = p_ref[0] + p_ref[1] + m_ref[...]
    o_ref[...] = dinv[:, None] * s + b_ref[...]


def kernel(x, edge_index, W, b):
    row = edge_index[0].astype(jnp.int32)
    col = edge_index[1].astype(jnp.int32)

    hist = _deg_kernel(row, col).T

    m = pl.pallas_call(
        _norm_body,
        grid=(GRID,),
        in_specs=[
            pl.BlockSpec((BR, D), lambda r: (r, 0)),
            pl.BlockSpec((D, D), lambda r: (0, 0)),
            pl.BlockSpec((BR, NW), lambda r: (r, 0)),
        ],
        out_specs=pl.BlockSpec((BR, D), lambda r: (r, 0)),
        out_shape=jax.ShapeDtypeStruct((N, D), jnp.float32),
    )(x, W, hist)

    npad = E_PAD - E
    row_p = jnp.concatenate([row, jnp.zeros((npad,), jnp.int32)])
    col_p = jnp.concatenate([col, jnp.full((npad,), N, jnp.int32)])
    partials = _edge_kernel(m, row_p.reshape(NW, NCHUNK, K),
                            col_p.reshape(NW, NCHUNK, K))

    out = pl.pallas_call(
        _final_body,
        grid=(GRID,),
        in_specs=[
            pl.BlockSpec((NC, BR, D), lambda r: (0, r, 0)),
            pl.BlockSpec((BR, D), lambda r: (r, 0)),
            pl.BlockSpec((BR, NW), lambda r: (r, 0)),
            pl.BlockSpec((1, D), lambda r: (0, 0)),
        ],
        out_specs=pl.BlockSpec((BR, D), lambda r: (r, 0)),
        out_shape=jax.ShapeDtypeStruct((N, D), jnp.float32),
    )(partials, m, hist, b[None, :])

    return out


# speedup vs baseline: 1.0013x; 1.0013x over previous
import functools

import jax
import jax.numpy as jnp
from jax import lax
from jax.experimental import pallas as pl
from jax.experimental.pallas import tpu as pltpu
from jax.experimental.pallas import tpu_sc as plsc

N = 10000
E = 320000
D = 128

NC = 2
NS = 16
NW = NC * NS
EPW = E // NW
K = 128
E_PAD = 327680
EPP = E_PAD // NW
NCHUNK = EPP // K
ACC_ROWS = 10112
ROWS_PT = N // NS

BR = 400
GRID = N // BR

_mesh = plsc.VectorSubcoreMesh(core_axis_name="c", subcore_axis_name="s")


@functools.partial(
    pl.kernel,
    mesh=_mesh,
    compiler_params=pltpu.CompilerParams(needs_layout_passes=False),
    out_type=jax.ShapeDtypeStruct((NW, N), jnp.float32),
    scratch_types=[
        pltpu.VMEM((N,), jnp.float32),
        pltpu.VMEM((EPW,), jnp.int32),
        pltpu.VMEM((EPW,), jnp.int32),
    ],
)
def _deg_kernel(row_hbm, col_hbm, hist_hbm, hist_v, row_v, col_v):
    wid = lax.axis_index("s") * NC + lax.axis_index("c")
    base = wid * EPW
    pltpu.sync_copy(row_hbm.at[pl.ds(base, EPW)], row_v)
    pltpu.sync_copy(col_hbm.at[pl.ds(base, EPW)], col_v)

    zv = jnp.zeros((16,), jnp.float32)

    def zbody(i, t):
        hist_v[pl.ds(i * 16, 16)] = zv
        return t

    lax.fori_loop(0, N // 16, zbody, 0)

    ones = jnp.ones((16,), jnp.float32)
    zero = jnp.zeros((16,), jnp.float32)

    def ebody(i, t):
        r = row_v[pl.ds(i * 16, 16)]
        c = col_v[pl.ds(i * 16, 16)]
        val = jnp.where(r == c, zero, ones)
        plsc.addupdate_scatter(hist_v, [r], val)
        return t

    lax.fori_loop(0, EPW // 16, ebody, 0)
    pltpu.sync_copy(hist_v, hist_hbm.at[wid])


def _norm_body(x_ref, w_ref, hist_ref, m_ref):
    h = jnp.dot(x_ref[...], w_ref[...], preferred_element_type=jnp.float32)
    deg = jnp.sum(hist_ref[...], axis=1) + 1.0
    dinv = lax.rsqrt(deg)
    m_ref[...] = h * dinv[:, None]


NB = 2
PASSES = 2
CPP = NCHUNK // PASSES
GRP = CPP // NB
ZPT = ACC_ROWS // NS


@functools.partial(
    pl.kernel,
    mesh=_mesh,
    compiler_params=pltpu.CompilerParams(needs_layout_passes=False),
    out_type=jax.ShapeDtypeStruct((NC, ACC_ROWS, D), jnp.float32),
    scratch_types=[
        pltpu.VMEM_SHARED((ACC_ROWS, D), jnp.float32),
        pltpu.VMEM((CPP, K), jnp.int32),
        pltpu.VMEM((CPP, K), jnp.int32),
        [pltpu.VMEM((K, D), jnp.float32)] * NB,
        [pltpu.SemaphoreType.DMA] * NB,
        pltpu.SemaphoreType.DMA,
    ],
)
def _edge_kernel(m_hbm, row_hbm, col_hbm, outp_hbm,
                 acc_s, row_v, col_v, gbufs, gsems, isem):
    cid = lax.axis_index("c")
    sid = lax.axis_index("s")
    wid = sid * NC + cid

    idx_row = pltpu.async_copy(row_hbm.at[wid, pl.ds(0, CPP)], row_v, isem)
    idx_col = pltpu.async_copy(col_hbm.at[wid, pl.ds(0, CPP)], col_v, isem)

    zv = jnp.zeros((16,), jnp.float32)

    def zb(i, t):
        gbufs[0][i // 8, pl.ds((i % 8) * 16, 16)] = zv
        return t

    lax.fori_loop(0, K * (D // 16), zb, 0)

    def zacc(i, t):
        pltpu.sync_copy(gbufs[0], acc_s.at[pl.ds(sid * ZPT + i * K, K)])
        return t

    lax.fori_loop(0, ZPT // K, zacc, 0)
    ztail = ZPT % K
    if ztail:
        pltpu.sync_copy(gbufs[0].at[pl.ds(0, ztail)],
                        acc_s.at[pl.ds(sid * ZPT + (ZPT // K) * K, ztail)])

    idx_row.wait()
    idx_col.wait()
    plsc.subcore_barrier()

    trash = jnp.full((16,), N, jnp.int32)

    for p in range(PASSES):
        if p > 0:
            pltpu.async_copy(
                row_hbm.at[wid, pl.ds(p * CPP, CPP)], row_v, isem).wait()
            pltpu.async_copy(
                col_hbm.at[wid, pl.ds(p * CPP, CPP)], col_v, isem).wait()

        def adj(ci, t):
            for j in range(K // 16):
                r = row_v[ci, pl.ds(j * 16, 16)]
                c = col_v[ci, pl.ds(j * 16, 16)]
                col_v[ci, pl.ds(j * 16, 16)] = jnp.where(r == c, trash, c)
            return t

        lax.fori_loop(0, CPP, adj, 0)

        for b in range(NB):
            pltpu.async_copy(m_hbm.at[row_v.at[b]], gbufs[b], gsems[b])

        def group(g, t):
            ci0 = g * NB
            for b in range(NB):
                ci = ci0 + b
                pltpu.make_async_copy(m_hbm.at[row_v.at[0]], gbufs[b],
                                      gsems[b]).wait()
                pltpu.sync_copy(gbufs[b], acc_s.at[col_v.at[ci]], add=True)
                pltpu.async_copy(m_hbm.at[row_v.at[ci + NB]], gbufs[b],
                                 gsems[b])
            return t

        lax.fori_loop(0, GRP - 1, group, 0)

        for b in range(NB):
            ci = (GRP - 1) * NB + b
            pltpu.make_async_copy(m_hbm.at[row_v.at[0]], gbufs[b],
                                  gsems[b]).wait()
            pltpu.sync_copy(gbufs[b], acc_s.at[col_v.at[ci]], add=True)

    plsc.subcore_barrier()
    pltpu.sync_copy(acc_s.at[pl.ds(sid * ZPT, ZPT)],
                    outp_hbm.at[cid, pl.ds(sid * ZPT, ZPT)])


def _final_body(p_ref, m_ref, hist_ref, b_ref, o_ref):
    deg = jnp.sum(hist_ref[...], axis=1) + 1.0
    dinv = lax.rsqrt(deg)
    s = p_ref[0] + p_ref[1] + m_ref[...]
    o_ref[...] = dinv[:, None] * s + b_ref[...]


def kernel(x, edge_index, W, b):
    row = edge_index[0].astype(jnp.int32)
    col = edge_index[1].astype(jnp.int32)

    hist = _deg_kernel(row, col).T

    m = pl.pallas_call(
        _norm_body,
        grid=(GRID,),
        in_specs=[
            pl.BlockSpec((BR, D), lambda r: (r, 0)),
            pl.BlockSpec((D, D), lambda r: (0, 0)),
            pl.BlockSpec((BR, NW), lambda r: (r, 0)),
        ],
        out_specs=pl.BlockSpec((BR, D), lambda r: (r, 0)),
        out_shape=jax.ShapeDtypeStruct((N, D), jnp.float32),
    )(x, W, hist)

    npad = E_PAD - E
    row_p = jnp.concatenate([row, jnp.zeros((npad,), jnp.int32)])
    trash_cols = N + jnp.arange(npad, dtype=jnp.int32) % (ACC_ROWS - N)
    col_p = jnp.concatenate([col, trash_cols])
    partials = _edge_kernel(m, row_p.reshape(NW, NCHUNK, K),
                            col_p.reshape(NW, NCHUNK, K))

    out = pl.pallas_call(
        _final_body,
        grid=(GRID,),
        in_specs=[
            pl.BlockSpec((NC, BR, D), lambda r: (0, r, 0)),
            pl.BlockSpec((BR, D), lambda r: (r, 0)),
            pl.BlockSpec((BR, NW), lambda r: (r, 0)),
            pl.BlockSpec((1, D), lambda r: (0, 0)),
        ],
        out_specs=pl.BlockSpec((BR, D), lambda r: (r, 0)),
        out_shape=jax.ShapeDtypeStruct((N, D), jnp.float32),
    )(partials, m, hist, b[None, :])

    return out


# speedup vs baseline: 2.2777x; 2.2748x over previous
import functools

import jax
import jax.numpy as jnp
from jax import lax
from jax.experimental import pallas as pl
from jax.experimental.pallas import tpu as pltpu
from jax.experimental.pallas import tpu_sc as plsc

N = 10000
E = 320000
D = 128

NC = 2
NS = 16
NW = NC * NS
EPW = E // NW
K = 80
NCHUNK = EPW // K
ACC_ROWS = 10240
ZPT = ACC_ROWS // NS

BR = 400
GRID = N // BR

_mesh = plsc.VectorSubcoreMesh(core_axis_name="c", subcore_axis_name="s")


@functools.partial(
    pl.kernel,
    mesh=_mesh,
    compiler_params=pltpu.CompilerParams(needs_layout_passes=False),
    out_type=jax.ShapeDtypeStruct((NW, N), jnp.float32),
    scratch_types=[
        pltpu.VMEM((N,), jnp.float32),
        pltpu.VMEM((EPW,), jnp.int32),
        pltpu.VMEM((EPW,), jnp.int32),
    ],
)
def _deg_kernel(row_hbm, col_hbm, hist_hbm, hist_v, row_v, col_v):
    wid = lax.axis_index("s") * NC + lax.axis_index("c")
    base = wid * EPW
    pltpu.sync_copy(row_hbm.at[pl.ds(base, EPW)], row_v)
    pltpu.sync_copy(col_hbm.at[pl.ds(base, EPW)], col_v)

    zv = jnp.zeros((16,), jnp.float32)

    def zbody(i, t):
        hist_v[pl.ds(i * 16, 16)] = zv
        return t

    lax.fori_loop(0, N // 16, zbody, 0)

    ones = jnp.ones((16,), jnp.float32)
    zero = jnp.zeros((16,), jnp.float32)

    def ebody(i, t):
        r = row_v[pl.ds(i * 16, 16)]
        c = col_v[pl.ds(i * 16, 16)]
        val = jnp.where(r == c, zero, ones)
        plsc.addupdate_scatter(hist_v, [r], val)
        return t

    lax.fori_loop(0, EPW // 16, ebody, 0)
    pltpu.sync_copy(hist_v, hist_hbm.at[wid])


def _norm_body(x_ref, w_ref, hist_ref, m_ref):
    h = jnp.dot(x_ref[...], w_ref[...], preferred_element_type=jnp.float32)
    deg = jnp.sum(hist_ref[...], axis=1) + 1.0
    dinv = lax.rsqrt(deg)
    m_ref[...] = h * dinv[:, None]


@functools.partial(
    pl.kernel,
    mesh=_mesh,
    compiler_params=pltpu.CompilerParams(needs_layout_passes=False),
    out_type=jax.ShapeDtypeStruct((NC, ACC_ROWS, D), jnp.float32),
    scratch_types=[
        pltpu.VMEM_SHARED((ACC_ROWS, D), jnp.float32),
        pltpu.VMEM((NCHUNK, K), jnp.int32),
        pltpu.VMEM((NCHUNK, K), jnp.int32),
        pltpu.VMEM((K, D), jnp.float32),
        pltpu.SemaphoreType.DMA,
    ],
)
def _edge_kernel(m_hbm, row_hbm, col_hbm, outp_hbm,
                 acc_s, row_v, col_v, gbuf, isem):
    cid = lax.axis_index("c")
    sid = lax.axis_index("s")
    wid = sid * NC + cid

    idx_row = pltpu.async_copy(row_hbm.at[wid], row_v, isem)
    idx_col = pltpu.async_copy(col_hbm.at[wid], col_v, isem)

    zv = jnp.zeros((16,), jnp.float32)

    def zb(i, t):
        gbuf[i // 8, pl.ds((i % 8) * 16, 16)] = zv
        return t

    lax.fori_loop(0, K * (D // 16), zb, 0)

    def zacc(i, t):
        pltpu.sync_copy(gbuf, acc_s.at[pl.ds(sid * ZPT + i * K, K)])
        return t

    lax.fori_loop(0, ZPT // K, zacc, 0)

    idx_row.wait()
    idx_col.wait()
    plsc.subcore_barrier()

    def chunk(ci, t):
        pltpu.sync_copy(m_hbm.at[row_v.at[ci]], gbuf)
        pltpu.sync_copy(gbuf, acc_s.at[col_v.at[ci]], add=True)
        return t

    lax.fori_loop(0, NCHUNK, chunk, 0)

    plsc.subcore_barrier()
    pltpu.sync_copy(acc_s.at[pl.ds(sid * ZPT, ZPT)],
                    outp_hbm.at[cid, pl.ds(sid * ZPT, ZPT)])


def _final_body(p_ref, m_ref, hist_ref, b_ref, o_ref):
    deg = jnp.sum(hist_ref[...], axis=1) + 1.0
    dinv = lax.rsqrt(deg)
    s = p_ref[0] + p_ref[1] + m_ref[...]
    o_ref[...] = dinv[:, None] * s + b_ref[...]


def kernel(x, edge_index, W, b):
    row = edge_index[0].astype(jnp.int32)
    col = edge_index[1].astype(jnp.int32)

    hist = _deg_kernel(row, col).T

    m = pl.pallas_call(
        _norm_body,
        grid=(GRID,),
        in_specs=[
            pl.BlockSpec((BR, D), lambda r: (r, 0)),
            pl.BlockSpec((D, D), lambda r: (0, 0)),
            pl.BlockSpec((BR, NW), lambda r: (r, 0)),
        ],
        out_specs=pl.BlockSpec((BR, D), lambda r: (r, 0)),
        out_shape=jax.ShapeDtypeStruct((N, D), jnp.float32),
    )(x, W, hist)

    trash = N + (jnp.arange(E, dtype=jnp.int32) % (ACC_ROWS - N))
    col_adj = jnp.where(row == col, trash, col)
    partials = _edge_kernel(m, row.reshape(NW, NCHUNK, K),
                            col_adj.reshape(NW, NCHUNK, K))

    out = pl.pallas_call(
        _final_body,
        grid=(GRID,),
        in_specs=[
            pl.BlockSpec((NC, BR, D), lambda r: (0, r, 0)),
            pl.BlockSpec((BR, D), lambda r: (r, 0)),
            pl.BlockSpec((BR, NW), lambda r: (r, 0)),
            pl.BlockSpec((1, D), lambda r: (0, 0)),
        ],
        out_specs=pl.BlockSpec((BR, D), lambda r: (r, 0)),
        out_shape=jax.ShapeDtypeStruct((N, D), jnp.float32),
    )(partials, m, hist, b[None, :])

    return out


# speedup vs baseline: 2.7464x; 1.2058x over previous
import functools

import jax
import jax.numpy as jnp
from jax import lax
from jax.experimental import pallas as pl
from jax.experimental.pallas import tpu as pltpu
from jax.experimental.pallas import tpu_sc as plsc

N = 10000
E = 320000
D = 128

NC = 2
NS = 16
NW = NC * NS
EPW = E // NW
K = 40
NCHUNK = EPW // K
NB = 2
GRP = (NCHUNK - 2) // NB
ACC_ROWS = 10240
ZPT = ACC_ROWS // NS

BR = 400
GRID = N // BR

_mesh = plsc.VectorSubcoreMesh(core_axis_name="c", subcore_axis_name="s")


@functools.partial(
    pl.kernel,
    mesh=_mesh,
    compiler_params=pltpu.CompilerParams(needs_layout_passes=False),
    out_type=jax.ShapeDtypeStruct((NW, N), jnp.float32),
    scratch_types=[
        pltpu.VMEM((N,), jnp.float32),
        pltpu.VMEM((EPW,), jnp.int32),
        pltpu.VMEM((EPW,), jnp.int32),
    ],
)
def _deg_kernel(row_hbm, col_hbm, hist_hbm, hist_v, row_v, col_v):
    wid = lax.axis_index("s") * NC + lax.axis_index("c")
    base = wid * EPW
    pltpu.sync_copy(row_hbm.at[pl.ds(base, EPW)], row_v)
    pltpu.sync_copy(col_hbm.at[pl.ds(base, EPW)], col_v)

    zv = jnp.zeros((16,), jnp.float32)

    def zbody(i, t):
        hist_v[pl.ds(i * 16, 16)] = zv
        return t

    lax.fori_loop(0, N // 16, zbody, 0)

    ones = jnp.ones((16,), jnp.float32)
    zero = jnp.zeros((16,), jnp.float32)

    def ebody(i, t):
        r = row_v[pl.ds(i * 16, 16)]
        c = col_v[pl.ds(i * 16, 16)]
        val = jnp.where(r == c, zero, ones)
        plsc.addupdate_scatter(hist_v, [r], val)
        return t

    lax.fori_loop(0, EPW // 16, ebody, 0)
    pltpu.sync_copy(hist_v, hist_hbm.at[wid])


def _norm_body(x_ref, w_ref, hist_ref, m_ref):
    h = jnp.dot(x_ref[...], w_ref[...], preferred_element_type=jnp.float32)
    deg = jnp.sum(hist_ref[...], axis=1) + 1.0
    dinv = lax.rsqrt(deg)
    m_ref[...] = h * dinv[:, None]


@functools.partial(
    pl.kernel,
    mesh=_mesh,
    compiler_params=pltpu.CompilerParams(needs_layout_passes=False),
    out_type=jax.ShapeDtypeStruct((NC, ACC_ROWS, D), jnp.float32),
    scratch_types=[
        pltpu.VMEM_SHARED((ACC_ROWS, D), jnp.float32),
        pltpu.VMEM((EPW,), jnp.int32),
        pltpu.VMEM((EPW,), jnp.int32),
        [pltpu.VMEM((K, D), jnp.float32)] * NB,
        [pltpu.SemaphoreType.DMA] * NB,
        pltpu.SemaphoreType.DMA,
    ],
)
def _edge_kernel(m_hbm, row_hbm, col_hbm, outp_hbm,
                 acc_s, row_v, col_v, gbufs, gsems, isem):
    cid = lax.axis_index("c")
    sid = lax.axis_index("s")
    wid = sid * NC + cid

    idx_row = pltpu.async_copy(row_hbm.at[wid], row_v, isem)
    idx_col = pltpu.async_copy(col_hbm.at[wid], col_v, isem)

    zv = jnp.zeros((16,), jnp.float32)

    def zb(i, t):
        gbufs[0][i // 8, pl.ds((i % 8) * 16, 16)] = zv
        return t

    lax.fori_loop(0, K * (D // 16), zb, 0)

    def zacc(i, t):
        pltpu.sync_copy(gbufs[0], acc_s.at[pl.ds(sid * ZPT + i * K, K)])
        return t

    lax.fori_loop(0, ZPT // K, zacc, 0)

    idx_row.wait()
    idx_col.wait()
    plsc.subcore_barrier()

    for b in range(NB):
        pltpu.async_copy(m_hbm.at[row_v.at[pl.ds(b * K, K)]],
                         gbufs[b], gsems[b])

    def group(g, t):
        for b in range(NB):
            ci = g * NB + b
            pltpu.make_async_copy(m_hbm.at[row_v.at[pl.ds(0, K)]],
                                  gbufs[b], gsems[b]).wait()
            pltpu.sync_copy(gbufs[b],
                            acc_s.at[col_v.at[pl.ds(ci * K, K)]], add=True)
            pltpu.async_copy(m_hbm.at[row_v.at[pl.ds((ci + NB) * K, K)]],
                             gbufs[b], gsems[b])
        return t

    lax.fori_loop(0, GRP, group, 0)

    c0 = GRP * NB
    for b in range(NB):
        pltpu.make_async_copy(m_hbm.at[row_v.at[pl.ds(0, K)]],
                              gbufs[b], gsems[b]).wait()
        pltpu.sync_copy(gbufs[b],
                        acc_s.at[col_v.at[pl.ds((c0 + b) * K, K)]],
                        add=True)

    plsc.subcore_barrier()
    pltpu.sync_copy(acc_s.at[pl.ds(sid * ZPT, ZPT)],
                    outp_hbm.at[cid, pl.ds(sid * ZPT, ZPT)])


def _final_body(p_ref, m_ref, hist_ref, b_ref, o_ref):
    deg = jnp.sum(hist_ref[...], axis=1) + 1.0
    dinv = lax.rsqrt(deg)
    s = p_ref[0] + p_ref[1] + m_ref[...]
    o_ref[...] = dinv[:, None] * s + b_ref[...]


def kernel(x, edge_index, W, b):
    row = edge_index[0].astype(jnp.int32)
    col = edge_index[1].astype(jnp.int32)

    hist = _deg_kernel(row, col).T

    m = pl.pallas_call(
        _norm_body,
        grid=(GRID,),
        in_specs=[
            pl.BlockSpec((BR, D), lambda r: (r, 0)),
            pl.BlockSpec((D, D), lambda r: (0, 0)),
            pl.BlockSpec((BR, NW), lambda r: (r, 0)),
        ],
        out_specs=pl.BlockSpec((BR, D), lambda r: (r, 0)),
        out_shape=jax.ShapeDtypeStruct((N, D), jnp.float32),
    )(x, W, hist)

    trash = N + (jnp.arange(E, dtype=jnp.int32) % (ACC_ROWS - N))
    col_adj = jnp.where(row == col, trash, col)
    partials = _edge_kernel(m, row.reshape(NW, EPW),
                            col_adj.reshape(NW, EPW))

    out = pl.pallas_call(
        _final_body,
        grid=(GRID,),
        in_specs=[
            pl.BlockSpec((NC, BR, D), lambda r: (0, r, 0)),
            pl.BlockSpec((BR, D), lambda r: (r, 0)),
            pl.BlockSpec((BR, NW), lambda r: (r, 0)),
            pl.BlockSpec((1, D), lambda r: (0, 0)),
        ],
        out_specs=pl.BlockSpec((BR, D), lambda r: (r, 0)),
        out_shape=jax.ShapeDtypeStruct((N, D), jnp.float32),
    )(partials, m, hist, b[None, :])

    return out


# speedup vs baseline: 3.3422x; 1.2169x over previous
import functools

import jax
import jax.numpy as jnp
from jax import lax
from jax.experimental import pallas as pl
from jax.experimental.pallas import tpu as pltpu
from jax.experimental.pallas import tpu_sc as plsc

N = 10000
E = 320000
D = 128

NC = 2
NS = 16
NW = NC * NS
EPW = E // NW
K = 40
NCHUNK = EPW // K
NB = 3
GRP = 82
ACC_ROWS = 10240
ZPT = ACC_ROWS // NS

BR = 400
GRID = N // BR

_mesh = plsc.VectorSubcoreMesh(core_axis_name="c", subcore_axis_name="s")


@functools.partial(
    pl.kernel,
    mesh=_mesh,
    compiler_params=pltpu.CompilerParams(needs_layout_passes=False),
    out_type=jax.ShapeDtypeStruct((NW, N), jnp.float32),
    scratch_types=[
        pltpu.VMEM((N,), jnp.float32),
        pltpu.VMEM((EPW,), jnp.int32),
        pltpu.VMEM((EPW,), jnp.int32),
    ],
)
def _deg_kernel(row_hbm, col_hbm, hist_hbm, hist_v, row_v, col_v):
    wid = lax.axis_index("s") * NC + lax.axis_index("c")
    base = wid * EPW
    pltpu.sync_copy(row_hbm.at[pl.ds(base, EPW)], row_v)
    pltpu.sync_copy(col_hbm.at[pl.ds(base, EPW)], col_v)

    zv = jnp.zeros((16,), jnp.float32)

    def zbody(i, t):
        hist_v[pl.ds(i * 16, 16)] = zv
        return t

    lax.fori_loop(0, N // 16, zbody, 0)

    ones = jnp.ones((16,), jnp.float32)
    zero = jnp.zeros((16,), jnp.float32)

    def ebody(i, t):
        r = row_v[pl.ds(i * 16, 16)]
        c = col_v[pl.ds(i * 16, 16)]
        val = jnp.where(r == c, zero, ones)
        plsc.addupdate_scatter(hist_v, [r], val)
        return t

    lax.fori_loop(0, EPW // 16, ebody, 0)
    pltpu.sync_copy(hist_v, hist_hbm.at[wid])


def _norm_body(x_ref, w_ref, hist_ref, m_ref):
    h = jnp.dot(x_ref[...], w_ref[...], preferred_element_type=jnp.float32)
    deg = jnp.sum(hist_ref[...], axis=1) + 1.0
    dinv = lax.rsqrt(deg)
    m_ref[...] = h * dinv[:, None]


@functools.partial(
    pl.kernel,
    mesh=_mesh,
    compiler_params=pltpu.CompilerParams(needs_layout_passes=False),
    out_type=jax.ShapeDtypeStruct((NC, ACC_ROWS, D), jnp.float32),
    scratch_types=[
        pltpu.VMEM_SHARED((ACC_ROWS, D), jnp.float32),
        pltpu.VMEM((EPW,), jnp.int32),
        pltpu.VMEM((EPW,), jnp.int32),
        [pltpu.VMEM((K, D), jnp.float32)] * NB,
        [pltpu.SemaphoreType.DMA] * NB,
        pltpu.SemaphoreType.DMA,
    ],
)
def _edge_kernel(m_hbm, row_hbm, col_hbm, outp_hbm,
                 acc_s, row_v, col_v, gbufs, gsems, isem):
    cid = lax.axis_index("c")
    sid = lax.axis_index("s")
    wid = sid * NC + cid

    idx_row = pltpu.async_copy(row_hbm.at[wid], row_v, isem)
    idx_col = pltpu.async_copy(col_hbm.at[wid], col_v, isem)

    zv = jnp.zeros((16,), jnp.float32)

    def zb(i, t):
        gbufs[0][i // 8, pl.ds((i % 8) * 16, 16)] = zv
        return t

    lax.fori_loop(0, K * (D // 16), zb, 0)

    for i in range(ZPT // K):
        pltpu.async_copy(gbufs[0], acc_s.at[pl.ds(sid * ZPT + i * K, K)],
                         gsems[i % NB])
    for i in range(ZPT // K):
        pltpu.make_async_copy(
            gbufs[0], acc_s.at[pl.ds(sid * ZPT + i * K, K)],
            gsems[i % NB]).wait()

    idx_row.wait()
    idx_col.wait()
    plsc.subcore_barrier()

    for b in range(NB):
        pltpu.async_copy(m_hbm.at[row_v.at[pl.ds(b * K, K)]],
                         gbufs[b], gsems[b])

    def group(g, t):
        for b in range(NB):
            ci = g * NB + b
            pltpu.make_async_copy(m_hbm.at[row_v.at[pl.ds(0, K)]],
                                  gbufs[b], gsems[b]).wait()
            pltpu.sync_copy(gbufs[b],
                            acc_s.at[col_v.at[pl.ds(ci * K, K)]], add=True)
            pltpu.async_copy(m_hbm.at[row_v.at[pl.ds((ci + NB) * K, K)]],
                             gbufs[b], gsems[b])
        return t

    lax.fori_loop(0, GRP, group, 0)

    for ci in range(GRP * NB, NCHUNK):
        b = ci % NB
        pltpu.make_async_copy(m_hbm.at[row_v.at[pl.ds(0, K)]],
                              gbufs[b], gsems[b]).wait()
        pltpu.sync_copy(gbufs[b],
                        acc_s.at[col_v.at[pl.ds(ci * K, K)]], add=True)
        if ci + NB < NCHUNK:
            pltpu.async_copy(m_hbm.at[row_v.at[pl.ds((ci + NB) * K, K)]],
                             gbufs[b], gsems[b])

    plsc.subcore_barrier()
    pltpu.sync_copy(acc_s.at[pl.ds(sid * ZPT, ZPT)],
                    outp_hbm.at[cid, pl.ds(sid * ZPT, ZPT)])


def _final_body(p_ref, m_ref, hist_ref, b_ref, o_ref):
    deg = jnp.sum(hist_ref[...], axis=1) + 1.0
    dinv = lax.rsqrt(deg)
    s = p_ref[0] + p_ref[1] + m_ref[...]
    o_ref[...] = dinv[:, None] * s + b_ref[...]


def kernel(x, edge_index, W, b):
    row = edge_index[0].astype(jnp.int32)
    col = edge_index[1].astype(jnp.int32)

    hist = _deg_kernel(row, col).T

    m = pl.pallas_call(
        _norm_body,
        grid=(GRID,),
        in_specs=[
            pl.BlockSpec((BR, D), lambda r: (r, 0)),
            pl.BlockSpec((D, D), lambda r: (0, 0)),
            pl.BlockSpec((BR, NW), lambda r: (r, 0)),
        ],
        out_specs=pl.BlockSpec((BR, D), lambda r: (r, 0)),
        out_shape=jax.ShapeDtypeStruct((N, D), jnp.float32),
    )(x, W, hist)

    trash = N + (jnp.arange(E, dtype=jnp.int32) % (ACC_ROWS - N))
    col_adj = jnp.where(row == col, trash, col)
    partials = _edge_kernel(m, row.reshape(NW, EPW),
                            col_adj.reshape(NW, EPW))

    out = pl.pallas_call(
        _final_body,
        grid=(GRID,),
        in_specs=[
            pl.BlockSpec((NC, BR, D), lambda r: (0, r, 0)),
            pl.BlockSpec((BR, D), lambda r: (r, 0)),
            pl.BlockSpec((BR, NW), lambda r: (r, 0)),
            pl.BlockSpec((1, D), lambda r: (0, 0)),
        ],
        out_specs=pl.BlockSpec((BR, D), lambda r: (r, 0)),
        out_shape=jax.ShapeDtypeStruct((N, D), jnp.float32),
    )(partials, m, hist, b[None, :])

    return out


# speedup vs baseline: 3.6690x; 1.0978x over previous
import functools

import jax
import jax.numpy as jnp
from jax import lax
from jax.experimental import pallas as pl
from jax.experimental.pallas import tpu as pltpu
from jax.experimental.pallas import tpu_sc as plsc

N = 10000
E = 320000
D = 128

NC = 2
NS = 16
NW = NC * NS
EPW = E // NW
K = 40
NCHUNK = EPW // K
NB = 4
GRP = 61
ACC_ROWS = 10240
ZPT = ACC_ROWS // NS

BR = 400
GRID = N // BR

_mesh = plsc.VectorSubcoreMesh(core_axis_name="c", subcore_axis_name="s")


@functools.partial(
    pl.kernel,
    mesh=_mesh,
    compiler_params=pltpu.CompilerParams(needs_layout_passes=False),
    out_type=jax.ShapeDtypeStruct((NW, N), jnp.float32),
    scratch_types=[
        pltpu.VMEM((N,), jnp.float32),
        pltpu.VMEM((EPW,), jnp.int32),
        pltpu.VMEM((EPW,), jnp.int32),
    ],
)
def _deg_kernel(row_hbm, col_hbm, hist_hbm, hist_v, row_v, col_v):
    wid = lax.axis_index("s") * NC + lax.axis_index("c")
    base = wid * EPW
    pltpu.sync_copy(row_hbm.at[pl.ds(base, EPW)], row_v)
    pltpu.sync_copy(col_hbm.at[pl.ds(base, EPW)], col_v)

    zv = jnp.zeros((16,), jnp.float32)

    def zbody(i, t):
        hist_v[pl.ds(i * 16, 16)] = zv
        return t

    lax.fori_loop(0, N // 16, zbody, 0)

    ones = jnp.ones((16,), jnp.float32)
    zero = jnp.zeros((16,), jnp.float32)

    def ebody(i, t):
        r = row_v[pl.ds(i * 16, 16)]
        c = col_v[pl.ds(i * 16, 16)]
        val = jnp.where(r == c, zero, ones)
        plsc.addupdate_scatter(hist_v, [r], val)
        return t

    lax.fori_loop(0, EPW // 16, ebody, 0)
    pltpu.sync_copy(hist_v, hist_hbm.at[wid])


def _norm_body(x_ref, w_ref, hist_ref, m_ref):
    h = jnp.dot(x_ref[...], w_ref[...], preferred_element_type=jnp.float32)
    deg = jnp.sum(hist_ref[...], axis=1) + 1.0
    dinv = lax.rsqrt(deg)
    m_ref[...] = h * dinv[:, None]


@functools.partial(
    pl.kernel,
    mesh=_mesh,
    compiler_params=pltpu.CompilerParams(needs_layout_passes=False),
    out_type=jax.ShapeDtypeStruct((NC, ACC_ROWS, D), jnp.float32),
    scratch_types=[
        pltpu.VMEM_SHARED((ACC_ROWS, D), jnp.float32),
        pltpu.VMEM((EPW,), jnp.int32),
        pltpu.VMEM((EPW,), jnp.int32),
        [pltpu.VMEM((K, D), jnp.float32)] * NB,
        [pltpu.SemaphoreType.DMA] * NB,
        pltpu.SemaphoreType.DMA,
    ],
)
def _edge_kernel(m_hbm, row_hbm, col_hbm, outp_hbm,
                 acc_s, row_v, col_v, gbufs, gsems, isem):
    cid = lax.axis_index("c")
    sid = lax.axis_index("s")
    wid = sid * NC + cid

    idx_row = pltpu.async_copy(row_hbm.at[wid], row_v, isem)
    idx_col = pltpu.async_copy(col_hbm.at[wid], col_v, isem)

    zv = jnp.zeros((16,), jnp.float32)

    def zb(i, t):
        gbufs[0][i // 8, pl.ds((i % 8) * 16, 16)] = zv
        return t

    lax.fori_loop(0, K * (D // 16), zb, 0)

    for i in range(ZPT // K):
        pltpu.async_copy(gbufs[0], acc_s.at[pl.ds(sid * ZPT + i * K, K)],
                         gsems[i % NB])
    for i in range(ZPT // K):
        pltpu.make_async_copy(
            gbufs[0], acc_s.at[pl.ds(sid * ZPT + i * K, K)],
            gsems[i % NB]).wait()

    idx_row.wait()
    idx_col.wait()
    plsc.subcore_barrier()

    for b in range(NB):
        pltpu.async_copy(m_hbm.at[row_v.at[pl.ds(b * K, K)]],
                         gbufs[b], gsems[b])

    def group(g, t):
        for b in range(NB):
            ci = g * NB + b
            pltpu.make_async_copy(m_hbm.at[row_v.at[pl.ds(0, K)]],
                                  gbufs[b], gsems[b]).wait()
            pltpu.sync_copy(gbufs[b],
                            acc_s.at[col_v.at[pl.ds(ci * K, K)]], add=True)
            pltpu.async_copy(m_hbm.at[row_v.at[pl.ds((ci + NB) * K, K)]],
                             gbufs[b], gsems[b])
        return t

    lax.fori_loop(0, GRP, group, 0)

    for ci in range(GRP * NB, NCHUNK):
        b = ci % NB
        pltpu.make_async_copy(m_hbm.at[row_v.at[pl.ds(0, K)]],
                              gbufs[b], gsems[b]).wait()
        pltpu.sync_copy(gbufs[b],
                        acc_s.at[col_v.at[pl.ds(ci * K, K)]], add=True)
        if ci + NB < NCHUNK:
            pltpu.async_copy(m_hbm.at[row_v.at[pl.ds((ci + NB) * K, K)]],
                             gbufs[b], gsems[b])

    plsc.subcore_barrier()
    pltpu.sync_copy(acc_s.at[pl.ds(sid * ZPT, ZPT)],
                    outp_hbm.at[cid, pl.ds(sid * ZPT, ZPT)])


def _final_body(p_ref, m_ref, hist_ref, b_ref, o_ref):
    deg = jnp.sum(hist_ref[...], axis=1) + 1.0
    dinv = lax.rsqrt(deg)
    s = p_ref[0] + p_ref[1] + m_ref[...]
    o_ref[...] = dinv[:, None] * s + b_ref[...]


def kernel(x, edge_index, W, b):
    row = edge_index[0].astype(jnp.int32)
    col = edge_index[1].astype(jnp.int32)

    hist = _deg_kernel(row, col).T

    m = pl.pallas_call(
        _norm_body,
        grid=(GRID,),
        in_specs=[
            pl.BlockSpec((BR, D), lambda r: (r, 0)),
            pl.BlockSpec((D, D), lambda r: (0, 0)),
            pl.BlockSpec((BR, NW), lambda r: (r, 0)),
        ],
        out_specs=pl.BlockSpec((BR, D), lambda r: (r, 0)),
        out_shape=jax.ShapeDtypeStruct((N, D), jnp.float32),
    )(x, W, hist)

    trash = N + (jnp.arange(E, dtype=jnp.int32) % (ACC_ROWS - N))
    col_adj = jnp.where(row == col, trash, col)
    partials = _edge_kernel(m, row.reshape(NW, EPW),
                            col_adj.reshape(NW, EPW))

    out = pl.pallas_call(
        _final_body,
        grid=(GRID,),
        in_specs=[
            pl.BlockSpec((NC, BR, D), lambda r: (0, r, 0)),
            pl.BlockSpec((BR, D), lambda r: (r, 0)),
            pl.BlockSpec((BR, NW), lambda r: (r, 0)),
            pl.BlockSpec((1, D), lambda r: (0, 0)),
        ],
        out_specs=pl.BlockSpec((BR, D), lambda r: (r, 0)),
        out_shape=jax.ShapeDtypeStruct((N, D), jnp.float32),
    )(partials, m, hist, b[None, :])

    return out


# speedup vs baseline: 3.7962x; 1.0347x over previous
import functools

import jax
import jax.numpy as jnp
from jax import lax
from jax.experimental import pallas as pl
from jax.experimental.pallas import tpu as pltpu
from jax.experimental.pallas import tpu_sc as plsc

N = 10000
E = 320000
D = 128

NC = 2
NS = 16
NW = NC * NS
EPW = E // NW
K = 40
NCHUNK = EPW // K
NB = 5
GRP = 49
ACC_ROWS = 10240
ZPT = ACC_ROWS // NS

BR = 400
GRID = N // BR

_mesh = plsc.VectorSubcoreMesh(core_axis_name="c", subcore_axis_name="s")


@functools.partial(
    pl.kernel,
    mesh=_mesh,
    compiler_params=pltpu.CompilerParams(needs_layout_passes=False),
    out_type=jax.ShapeDtypeStruct((NW, N), jnp.float32),
    scratch_types=[
        pltpu.VMEM((N,), jnp.float32),
        pltpu.VMEM((EPW,), jnp.int32),
        pltpu.VMEM((EPW,), jnp.int32),
    ],
)
def _deg_kernel(row_hbm, col_hbm, hist_hbm, hist_v, row_v, col_v):
    wid = lax.axis_index("s") * NC + lax.axis_index("c")
    base = wid * EPW
    pltpu.sync_copy(row_hbm.at[pl.ds(base, EPW)], row_v)
    pltpu.sync_copy(col_hbm.at[pl.ds(base, EPW)], col_v)

    zv = jnp.zeros((16,), jnp.float32)

    def zbody(i, t):
        hist_v[pl.ds(i * 16, 16)] = zv
        return t

    lax.fori_loop(0, N // 16, zbody, 0)

    ones = jnp.ones((16,), jnp.float32)
    zero = jnp.zeros((16,), jnp.float32)

    def ebody(i, t):
        r = row_v[pl.ds(i * 16, 16)]
        c = col_v[pl.ds(i * 16, 16)]
        val = jnp.where(r == c, zero, ones)
        plsc.addupdate_scatter(hist_v, [r], val)
        return t

    lax.fori_loop(0, EPW // 16, ebody, 0)
    pltpu.sync_copy(hist_v, hist_hbm.at[wid])


def _norm_body(x_ref, w_ref, hist_ref, m_ref):
    h = jnp.dot(x_ref[...], w_ref[...], preferred_element_type=jnp.float32)
    deg = jnp.sum(hist_ref[...], axis=1) + 1.0
    dinv = lax.rsqrt(deg)
    m_ref[...] = h * dinv[:, None]


@functools.partial(
    pl.kernel,
    mesh=_mesh,
    compiler_params=pltpu.CompilerParams(needs_layout_passes=False),
    out_type=jax.ShapeDtypeStruct((NC, ACC_ROWS, D), jnp.float32),
    scratch_types=[
        pltpu.VMEM_SHARED((ACC_ROWS, D), jnp.float32),
        pltpu.VMEM((EPW,), jnp.int32),
        pltpu.VMEM((EPW,), jnp.int32),
        [pltpu.VMEM((K, D), jnp.float32)] * NB,
        [pltpu.SemaphoreType.DMA] * NB,
        pltpu.SemaphoreType.DMA,
    ],
)
def _edge_kernel(m_hbm, row_hbm, col_hbm, outp_hbm,
                 acc_s, row_v, col_v, gbufs, gsems, isem):
    cid = lax.axis_index("c")
    sid = lax.axis_index("s")
    wid = sid * NC + cid

    idx_row = pltpu.async_copy(row_hbm.at[wid], row_v, isem)
    idx_col = pltpu.async_copy(col_hbm.at[wid], col_v, isem)

    zv = jnp.zeros((16,), jnp.float32)

    def zb(i, t):
        gbufs[0][i // 8, pl.ds((i % 8) * 16, 16)] = zv
        return t

    lax.fori_loop(0, K * (D // 16), zb, 0)

    for i in range(ZPT // K):
        pltpu.async_copy(gbufs[0], acc_s.at[pl.ds(sid * ZPT + i * K, K)],
                         gsems[i % NB])
    for i in range(ZPT // K):
        pltpu.make_async_copy(
            gbufs[0], acc_s.at[pl.ds(sid * ZPT + i * K, K)],
            gsems[i % NB]).wait()

    idx_row.wait()
    idx_col.wait()
    plsc.subcore_barrier()

    for b in range(NB):
        pltpu.async_copy(m_hbm.at[row_v.at[pl.ds(b * K, K)]],
                         gbufs[b], gsems[b])

    def group(g, t):
        for b in range(NB):
            ci = g * NB + b
            pltpu.make_async_copy(m_hbm.at[row_v.at[pl.ds(0, K)]],
                                  gbufs[b], gsems[b]).wait()
            pltpu.sync_copy(gbufs[b],
                            acc_s.at[col_v.at[pl.ds(ci * K, K)]], add=True)
            pltpu.async_copy(m_hbm.at[row_v.at[pl.ds((ci + NB) * K, K)]],
                             gbufs[b], gsems[b])
        return t

    lax.fori_loop(0, GRP, group, 0)

    for ci in range(GRP * NB, NCHUNK):
        b = ci % NB
        pltpu.make_async_copy(m_hbm.at[row_v.at[pl.ds(0, K)]],
                              gbufs[b], gsems[b]).wait()
        pltpu.sync_copy(gbufs[b],
                        acc_s.at[col_v.at[pl.ds(ci * K, K)]], add=True)
        if ci + NB < NCHUNK:
            pltpu.async_copy(m_hbm.at[row_v.at[pl.ds((ci + NB) * K, K)]],
                             gbufs[b], gsems[b])

    plsc.subcore_barrier()
    pltpu.sync_copy(acc_s.at[pl.ds(sid * ZPT, ZPT)],
                    outp_hbm.at[cid, pl.ds(sid * ZPT, ZPT)])


def _final_body(p_ref, m_ref, hist_ref, b_ref, o_ref):
    deg = jnp.sum(hist_ref[...], axis=1) + 1.0
    dinv = lax.rsqrt(deg)
    s = p_ref[0] + p_ref[1] + m_ref[...]
    o_ref[...] = dinv[:, None] * s + b_ref[...]


def kernel(x, edge_index, W, b):
    row = edge_index[0].astype(jnp.int32)
    col = edge_index[1].astype(jnp.int32)

    hist = _deg_kernel(row, col).T

    m = pl.pallas_call(
        _norm_body,
        grid=(GRID,),
        in_specs=[
            pl.BlockSpec((BR, D), lambda r: (r, 0)),
            pl.BlockSpec((D, D), lambda r: (0, 0)),
            pl.BlockSpec((BR, NW), lambda r: (r, 0)),
        ],
        out_specs=pl.BlockSpec((BR, D), lambda r: (r, 0)),
        out_shape=jax.ShapeDtypeStruct((N, D), jnp.float32),
    )(x, W, hist)

    trash = N + (jnp.arange(E, dtype=jnp.int32) % (ACC_ROWS - N))
    col_adj = jnp.where(row == col, trash, col)
    partials = _edge_kernel(m, row.reshape(NW, EPW),
                            col_adj.reshape(NW, EPW))

    out = pl.pallas_call(
        _final_body,
        grid=(GRID,),
        in_specs=[
            pl.BlockSpec((NC, BR, D), lambda r: (0, r, 0)),
            pl.BlockSpec((BR, D), lambda r: (r, 0)),
            pl.BlockSpec((BR, NW), lambda r: (r, 0)),
            pl.BlockSpec((1, D), lambda r: (0, 0)),
        ],
        out_specs=pl.BlockSpec((BR, D), lambda r: (r, 0)),
        out_shape=jax.ShapeDtypeStruct((N, D), jnp.float32),
    )(partials, m, hist, b[None, :])

    return out
